# baseline (speedup 1.0000x reference)
import functools

import jax
import jax.numpy as jnp
from jax import lax
from jax.experimental import pallas as pl
from jax.experimental.pallas import tpu as pltpu
from jax.experimental.pallas import tpu_sc as plsc

B = 16384
D = 32
NC = 2
NS = 16
L = 16
NW = NC * NS
BPW = B // NW
IDXW = 128
NIDX = BPW // IDXW
NGROUPS = BPW // L
GLOBAL_MEAN = 3.82

_mesh = plsc.VectorSubcoreMesh(core_axis_name="c", subcore_axis_name="s")


@functools.partial(
    pl.kernel,
    mesh=_mesh,
    compiler_params=pltpu.CompilerParams(
        needs_layout_passes=False, use_tc_tiling_on_sc=False),
    out_type=jax.ShapeDtypeStruct((B,), jnp.float32),
    scratch_types=[
        pltpu.VMEM((NIDX, IDXW), jnp.int32),
        pltpu.VMEM((NIDX, IDXW), jnp.int32),
        pltpu.VMEM((BPW, D), jnp.float32),
        pltpu.VMEM((BPW, D), jnp.float32),
        pltpu.VMEM((BPW,), jnp.float32),
        pltpu.VMEM((BPW,), jnp.float32),
        pltpu.VMEM((BPW,), jnp.float32),
        pltpu.SemaphoreType.DMA,
    ],
)
def _svdpp_sc(sid_hbm, pid_hbm, sf_hbm, pf_hbm, sb_hbm, pb_hbm, out_hbm,
              sid_v, pid_v, srows_v, prows_v, sb_v, pb_v, out_v, sem):
    wid = lax.axis_index("s") * NC + lax.axis_index("c")
    base = wid * BPW

    pltpu.sync_copy(sid_hbm.at[wid], sid_v)
    pltpu.sync_copy(pid_hbm.at[wid], pid_v)

    copies = []
    for j in range(NIDX):
        rows = pl.ds(j * IDXW, IDXW)
        copies.append(pltpu.async_copy(sf_hbm.at[sid_v.at[j]], srows_v.at[rows], sem))
        copies.append(pltpu.async_copy(pf_hbm.at[pid_v.at[j]], prows_v.at[rows], sem))
        copies.append(pltpu.async_copy(sb_hbm.at[sid_v.at[j]], sb_v.at[rows], sem))
        copies.append(pltpu.async_copy(pb_hbm.at[pid_v.at[j]], pb_v.at[rows], sem))
    for c in copies:
        c.wait()

    lanes = lax.iota(jnp.int32, L)

    def group_body(g, carry):
        rows = g * L + lanes
        acc = sb_v[pl.ds(g * L, L)] + pb_v[pl.ds(g * L, L)]
        for d in range(D):
            cols = jnp.full((L,), d, dtype=jnp.int32)
            sv = plsc.load_gather(srows_v, [rows, cols])
            pv = plsc.load_gather(prows_v, [rows, cols])
            acc = acc + sv * pv
        out_v[pl.ds(g * L, L)] = acc + jnp.float32(GLOBAL_MEAN)
        return carry

    lax.fori_loop(0, NGROUPS, group_body, 0)

    pltpu.sync_copy(out_v, out_hbm.at[pl.ds(base, BPW)])


def kernel(scientist_ids, paper_ids, scientist_factors, paper_factors,
           scientist_bias, paper_bias):
    sid = scientist_ids.reshape(NW, NIDX, IDXW)
    pid = paper_ids.reshape(NW, NIDX, IDXW)
    sb = scientist_bias.reshape(-1)
    pb = paper_bias.reshape(-1)
    return _svdpp_sc(sid, pid, scientist_factors, paper_factors, sb, pb)
